# baseline (speedup 1.0000x reference)
import functools

import jax
import jax.numpy as jnp
from jax import lax
from jax.experimental import pallas as pl
from jax.experimental.pallas import tpu as pltpu
from jax.experimental.pallas import tpu_sc as plsc

_CHUNK = 80
_DPAD = 128
_NHALF = 2


@functools.lru_cache(maxsize=None)
def _build_gather(V, D, B):
    info = plsc.get_sparse_core_info()
    NC, NS = info.num_cores, info.num_subcores
    NW = NC * NS
    assert B % (NW * _NHALF * _CHUNK) == 0, (B, NW)
    assert V % (NS * 8) == 0, (V, NS)
    b_per_w = B // NW
    b_half = b_per_w // _NHALF
    n_chunks = b_half // _CHUNK
    assert (n_chunks - 4) % 3 == 0, n_chunks
    mesh = plsc.VectorSubcoreMesh(core_axis_name="c", subcore_axis_name="s")

    @functools.partial(
        pl.kernel,
        mesh=mesh,
        out_type=jax.ShapeDtypeStruct((B, _DPAD), jnp.float32),
        scratch_types=[
            pltpu.VMEM_SHARED((V, _DPAD), jnp.float32),
            pltpu.VMEM((b_half,), jnp.int32),
            pltpu.VMEM((_CHUNK, _DPAD), jnp.float32),
            pltpu.VMEM((_CHUNK, _DPAD), jnp.float32),
            pltpu.VMEM((_CHUNK, _DPAD), jnp.float32),
            pltpu.SemaphoreType.DMA,
            pltpu.SemaphoreType.DMA,
            pltpu.SemaphoreType.DMA,
            pltpu.SemaphoreType.DMA,
        ],
    )
    def gather_kernel(table_hbm, idx_hbm, out_hbm, tab_s, idx_v,
                      rows0, rows1, rows2, sg, so0, so1, so2):
        sid = lax.axis_index("s")
        wid = sid * NC + lax.axis_index("c")
        base = wid * b_per_w

        v_slab = V // NS
        slab = sid * v_slab
        pltpu.sync_copy(table_hbm.at[pl.ds(slab, v_slab)],
                        tab_s.at[pl.ds(slab, v_slab)])

        plsc.subcore_barrier()

        bufs = (rows0, rows1, rows2)
        osems = (so0, so1, so2)

        def gather(c, b, src):
            pltpu.async_copy(
                src.at[idx_v.at[pl.ds(c * _CHUNK, _CHUNK)]],
                bufs[b], sg).wait()

        def start_out(hbase, c, b):
            pltpu.async_copy(
                bufs[b], out_hbm.at[pl.ds(hbase + c * _CHUNK, _CHUNK)],
                osems[b])

        def wait_out(b):
            pltpu.make_async_copy(
                bufs[b], out_hbm.at[pl.ds(base, _CHUNK)], osems[b]).wait()

        for h in range(_NHALF):
            hbase = base + h * b_half
            pltpu.sync_copy(idx_hbm.at[pl.ds(hbase, b_half)], idx_v)

            for j in range(4):
                b = j % 3
                if j >= 3:
                    wait_out(b)
                gather(j, b, tab_s)
                start_out(hbase, j, b)

            def body(p, carry, hbase=hbase):
                for j in range(3):
                    c = 4 + 3 * p + j
                    b = (1 + j) % 3
                    wait_out(b)
                    gather(c, b, tab_s)
                    start_out(hbase, c, b)
                return carry

            lax.fori_loop(0, (n_chunks - 4) // 3, body, 0)
            wait_out(0)
            wait_out(1)
            wait_out(2)

    return gather_kernel


def kernel(x, table):
    V, D = table.shape
    B = x.size
    idx = x.reshape(B).astype(jnp.int32)
    v_pad = -V % 128
    table_pad = jnp.pad(table, ((0, v_pad), (0, _DPAD - D)))
    out = _build_gather(V + v_pad, D, B)(table_pad, idx)
    return out.reshape(x.shape + (_DPAD,))[:, :, :D]


# speedup vs baseline: 1.0156x; 1.0156x over previous
import functools

import jax
import jax.numpy as jnp
from jax import lax
from jax.experimental import pallas as pl
from jax.experimental.pallas import tpu as pltpu
from jax.experimental.pallas import tpu_sc as plsc

_CHUNK = 128
_DPAD = 128
_NHALF = 2


@functools.lru_cache(maxsize=None)
def _build_gather(V, D, B):
    info = plsc.get_sparse_core_info()
    NC, NS = info.num_cores, info.num_subcores
    NW = NC * NS
    assert B % (NW * _NHALF * _CHUNK) == 0, (B, NW)
    assert V % (NS * 8) == 0, (V, NS)
    b_per_w = B // NW
    b_half = b_per_w // _NHALF
    n_chunks = b_half // _CHUNK
    assert n_chunks % 4 == 0, n_chunks
    mesh = plsc.VectorSubcoreMesh(core_axis_name="c", subcore_axis_name="s")

    @functools.partial(
        pl.kernel,
        mesh=mesh,
        out_type=jax.ShapeDtypeStruct((B, _DPAD), jnp.float32),
        scratch_types=[
            pltpu.VMEM_SHARED((V, _DPAD), jnp.float32),
            pltpu.VMEM((b_half,), jnp.int32),
            pltpu.VMEM((_CHUNK, _DPAD), jnp.float32),
            pltpu.VMEM((_CHUNK, _DPAD), jnp.float32),
            pltpu.SemaphoreType.DMA,
            pltpu.SemaphoreType.DMA,
            pltpu.SemaphoreType.DMA,
        ],
    )
    def gather_kernel(table_hbm, idx_hbm, out_hbm, tab_s, idx_v,
                      rows0, rows1, sg, so0, so1):
        sid = lax.axis_index("s")
        wid = sid * NC + lax.axis_index("c")
        base = wid * b_per_w

        v_slab = V // NS
        slab = sid * v_slab
        pltpu.sync_copy(table_hbm.at[pl.ds(slab, v_slab)],
                        tab_s.at[pl.ds(slab, v_slab)])

        plsc.subcore_barrier()

        bufs = (rows0, rows1)
        osems = (so0, so1)

        def gather(c, b, src):
            pltpu.async_copy(
                src.at[idx_v.at[pl.ds(c * _CHUNK, _CHUNK)]],
                bufs[b], sg).wait()

        def start_out(hbase, c, b):
            pltpu.async_copy(
                bufs[b], out_hbm.at[pl.ds(hbase + c * _CHUNK, _CHUNK)],
                osems[b])

        def wait_out(b):
            pltpu.make_async_copy(
                bufs[b], out_hbm.at[pl.ds(base, _CHUNK)], osems[b]).wait()

        for h in range(_NHALF):
            hbase = base + h * b_half
            pltpu.sync_copy(idx_hbm.at[pl.ds(hbase, b_half)], idx_v)

            for j in range(4):
                gather(j, j % 2, tab_s)
                if j < 2:
                    start_out(hbase, j, j)
                else:
                    wait_out(j % 2)
                    start_out(hbase, j, j % 2)

            def body(p, carry, hbase=hbase):
                for j in range(4):
                    b = j % 2
                    c = 4 * p + j
                    wait_out(b)
                    gather(c, b, tab_s)
                    start_out(hbase, c, b)
                return carry

            lax.fori_loop(1, n_chunks // 4, body, 0)
            wait_out(0)
            wait_out(1)

    return gather_kernel


def kernel(x, table):
    V, D = table.shape
    B = x.size
    idx = x.reshape(B).astype(jnp.int32)
    v_pad = -V % 128
    table_pad = jnp.pad(table, ((0, v_pad), (0, _DPAD - D)))
    out = _build_gather(V + v_pad, D, B)(table_pad, idx)
    return out.reshape(x.shape + (_DPAD,))[:, :, :D]
